# baseline (speedup 1.0000x reference)
import functools

import jax
import jax.numpy as jnp
from jax import lax
from jax.experimental import pallas as pl
from jax.experimental.pallas import tpu as pltpu
from jax.experimental.pallas import tpu_sc as plsc

_V = 8192
_D = 8192
_B = 4096
_NW = 32
_R = 4
_CPW = (_B // _NW) // _R

_mesh = plsc.VectorSubcoreMesh(core_axis_name="c", subcore_axis_name="s")


@functools.partial(
    pl.kernel,
    mesh=_mesh,
    out_type=jax.ShapeDtypeStruct((_B // _R, _R, _D), jnp.float32),
    scratch_types=[
        pltpu.VMEM((_CPW, _R), jnp.int32),
        pltpu.VMEM((_R, _D), jnp.float32),
        pltpu.VMEM((_R, _D), jnp.float32),
        pltpu.SemaphoreType.DMA,
        pltpu.SemaphoreType.DMA,
        pltpu.SemaphoreType.DMA,
        pltpu.SemaphoreType.DMA,
    ],
)
def _sc_gather(table_hbm, idx_hbm, out_hbm, idx_v, buf0, buf1,
               gsem0, gsem1, ssem0, ssem1):
    wid = lax.axis_index("s") * 2 + lax.axis_index("c")
    pltpu.sync_copy(idx_hbm.at[wid], idx_v)
    cbase = wid * _CPW

    pltpu.async_copy(table_hbm.at[idx_v.at[0]], buf0, gsem0)
    pltpu.async_copy(table_hbm.at[idx_v.at[1]], buf1, gsem1)

    def body(i, _):
        c = i * 2
        pltpu.make_async_copy(table_hbm.at[idx_v.at[c]], buf0, gsem0).wait()
        pltpu.async_copy(buf0, out_hbm.at[cbase + c], ssem0)

        pltpu.make_async_copy(table_hbm.at[idx_v.at[c + 1]], buf1, gsem1).wait()
        pltpu.async_copy(buf1, out_hbm.at[cbase + c + 1], ssem1)

        @pl.when(i < _CPW // 2 - 1)
        def _():
            pltpu.make_async_copy(buf0, out_hbm.at[cbase + c], ssem0).wait()
            pltpu.async_copy(table_hbm.at[idx_v.at[c + 2]], buf0, gsem0)
            pltpu.make_async_copy(buf1, out_hbm.at[cbase + c + 1], ssem1).wait()
            pltpu.async_copy(table_hbm.at[idx_v.at[c + 3]], buf1, gsem1)

        return 0

    lax.fori_loop(0, _CPW // 2, body, 0)

    last = cbase + _CPW - 2
    pltpu.make_async_copy(buf0, out_hbm.at[last], ssem0).wait()
    pltpu.make_async_copy(buf1, out_hbm.at[last + 1], ssem1).wait()


def kernel(idx, table):
    idx3 = idx.reshape(_NW, _CPW, _R).astype(jnp.int32)
    out = _sc_gather(table, idx3)
    return out.reshape(idx.shape[0], idx.shape[1], _D)
